# baseline (speedup 1.0000x reference)
import functools

import jax
import jax.numpy as jnp
from jax import lax
from jax.experimental import pallas as pl
from jax.experimental.pallas import tpu as pltpu
from jax.experimental.pallas import tpu_sc as plsc

NC, NS, L = 2, 16, 16
NW = NC * NS
B = 16384
F = 16
BPW = B // NW
G = BPW // L


def _lfm_body(users, items, ub_hbm, ib_hbm, ue_hbm, ie_hbm, out_hbm,
              idx_u, idx_i, ue_s, ie_s, ub_s, ib_s, out_s,
              sem_u, sem_i, sem_ub, sem_ib):
  wid = lax.axis_index("s") * NC + lax.axis_index("c")
  base = wid * BPW

  pltpu.sync_copy(users.at[pl.ds(base, BPW)], idx_u)
  pltpu.sync_copy(items.at[pl.ds(base, BPW)], idx_i)

  cu = pltpu.async_copy(ue_hbm.at[idx_u], ue_s, sem_u)
  ci = pltpu.async_copy(ie_hbm.at[idx_i], ie_s, sem_i)
  cub = pltpu.async_copy(ub_hbm.at[idx_u], ub_s, sem_ub)
  cib = pltpu.async_copy(ib_hbm.at[idx_i], ib_s, sem_ib)
  cu.wait()
  ci.wait()
  cub.wait()
  cib.wait()

  lane = lax.iota(jnp.int32, L)

  def group(g, carry):
    off = g * L
    rows = off + lane
    acc = ub_s[pl.ds(off, L)] + ib_s[pl.ds(off, L)]
    for f in range(F):
      col = jnp.full((L,), f, jnp.int32)
      acc = acc + (plsc.load_gather(ue_s, [rows, col]) *
                   plsc.load_gather(ie_s, [rows, col]))
    out_s[pl.ds(off, L)] = acc
    return carry

  lax.fori_loop(0, G, group, 0)
  pltpu.sync_copy(out_s, out_hbm.at[pl.ds(base, BPW)])


@functools.partial(jax.jit, static_argnames=())
def _lfm(users, items, ub, ib, ue, ie):
  mesh = plsc.VectorSubcoreMesh(
      core_axis_name="c", subcore_axis_name="s",
      num_cores=NC, num_subcores=NS)
  run = pl.kernel(
      _lfm_body,
      out_type=jax.ShapeDtypeStruct((B,), jnp.float32),
      mesh=mesh,
      compiler_params=pltpu.CompilerParams(needs_layout_passes=False,
                                           use_tc_tiling_on_sc=False),
      scratch_types=[
          pltpu.VMEM((BPW,), jnp.int32),
          pltpu.VMEM((BPW,), jnp.int32),
          pltpu.VMEM((BPW, F), jnp.float32),
          pltpu.VMEM((BPW, F), jnp.float32),
          pltpu.VMEM((BPW,), jnp.float32),
          pltpu.VMEM((BPW,), jnp.float32),
          pltpu.VMEM((BPW,), jnp.float32),
          pltpu.SemaphoreType.DMA,
          pltpu.SemaphoreType.DMA,
          pltpu.SemaphoreType.DMA,
          pltpu.SemaphoreType.DMA,
      ],
  )
  return run(users, items, ub, ib, ue, ie)


def kernel(users, items, user_biases, item_biases, user_embeddings,
           item_embeddings):
  users = users.astype(jnp.int32)
  items = items.astype(jnp.int32)
  ub = user_biases.reshape(-1)
  ib = item_biases.reshape(-1)
  return _lfm(users, items, ub, ib, user_embeddings, item_embeddings)
